# baseline (speedup 1.0000x reference)
import functools

import jax
import jax.numpy as jnp
from jax import lax
from jax.experimental import pallas as pl
from jax.experimental.pallas import tpu as pltpu
from jax.experimental.pallas import tpu_sc as plsc

BATCH = 4096
HIST = 200
EMBED_DIM = 64
NUM_CORES = 2
NUM_SUBCORES = 16
NW = NUM_CORES * NUM_SUBCORES
ROWS_W = BATCH // NW
HALF = HIST // 2
CB = 4
K = 2 * CB
NCHUNK = ROWS_W // CB

_mesh = plsc.VectorSubcoreMesh(core_axis_name="c", subcore_axis_name="s")


@functools.partial(
    pl.kernel,
    mesh=_mesh,
    out_type=jax.ShapeDtypeStruct((BATCH, HIST, EMBED_DIM), jnp.float32),
    scratch_types=[
        pltpu.VMEM((2 * ROWS_W, HALF), jnp.int32),
        pltpu.VMEM((2, CB, HIST, EMBED_DIM), jnp.float32),
        pltpu.SemaphoreType.DMA,
        pltpu.SemaphoreType.DMA,
        pltpu.SemaphoreType.DMA,
        pltpu.SemaphoreType.DMA,
    ],
    compiler_params=pltpu.CompilerParams(use_tc_tiling_on_sc=False),
)
def _embed(table_hbm, idx_hbm, out_hbm, idx_v, rows_v, g0, g1, o0, o1):
    wid = lax.axis_index("s") * NUM_CORES + lax.axis_index("c")
    base_b = wid * ROWS_W
    gsem = (g0, g1)
    osem = (o0, o1)

    def fire_g(c, b):
        for j in range(K):
            r, h = j // 2, j % 2
            pltpu.async_copy(
                table_hbm.at[idx_v.at[c * K + j]],
                rows_v.at[b].at[r].at[pl.ds(h * HALF, HALF)],
                gsem[b],
            )

    def wait_g(b):
        for _ in range(K):
            pltpu.make_async_copy(
                out_hbm.at[0].at[pl.ds(0, HALF)],
                rows_v.at[b].at[0].at[pl.ds(0, HALF)],
                gsem[b],
            ).wait()

    def fire_o(c, b):
        pltpu.async_copy(
            rows_v.at[b], out_hbm.at[pl.ds(base_b + c * CB, CB)], osem[b]
        )

    def drain_o(b):
        pltpu.make_async_copy(
            out_hbm.at[pl.ds(0, CB)], rows_v.at[b], osem[b]
        ).wait()

    pltpu.sync_copy(idx_hbm.at[pl.ds(wid * 2 * ROWS_W, 2 * ROWS_W)], idx_v)

    def step(c, b):
        wait_g(1 - b)
        fire_o(c - 1, 1 - b)
        drain_o(b)
        fire_g(c, b)

    fire_g(0, 0)
    wait_g(0)
    fire_o(0, 0)
    fire_g(1, 1)

    def body(g, carry):
        step(2 * g, 0)
        step(2 * g + 1, 1)
        return carry

    lax.fori_loop(1, NCHUNK // 2, body, 0)

    wait_g(1)
    fire_o(NCHUNK - 1, 1)
    drain_o(0)
    drain_o(1)


def kernel(vectors, table):
    idx = vectors.reshape(2 * BATCH, HALF)
    return _embed(table, idx)


# speedup vs baseline: 1.3963x; 1.3963x over previous
import functools

import jax
import jax.numpy as jnp
from jax import lax
from jax.experimental import pallas as pl
from jax.experimental.pallas import tpu as pltpu
from jax.experimental.pallas import tpu_sc as plsc

BATCH = 4096
HIST = 200
EMBED_DIM = 64
VOCAB = 1001
NUM_CORES = 2
NUM_SUBCORES = 16
NW = NUM_CORES * NUM_SUBCORES
ROWS_W = BATCH // NW
HALF = HIST // 2
CB = 2
K = 2 * CB
NCHUNK = ROWS_W // CB

_mesh = plsc.VectorSubcoreMesh(core_axis_name="c", subcore_axis_name="s")


@functools.partial(
    pl.kernel,
    mesh=_mesh,
    out_type=jax.ShapeDtypeStruct((BATCH, HIST, EMBED_DIM), jnp.float32),
    scratch_types=[
        pltpu.VMEM((2 * ROWS_W, HALF), jnp.int32),
        pltpu.VMEM((2, CB, HIST, EMBED_DIM), jnp.float32),
        pltpu.VMEM_SHARED((VOCAB, EMBED_DIM), jnp.float32),
        pltpu.SemaphoreType.DMA,
        pltpu.SemaphoreType.DMA,
        pltpu.SemaphoreType.DMA,
        pltpu.SemaphoreType.DMA,
    ],
    compiler_params=pltpu.CompilerParams(use_tc_tiling_on_sc=False),
)
def _embed(table_hbm, idx_hbm, out_hbm, idx_v, rows_v, tab_s, g0, g1, o0, o1):
    wid = lax.axis_index("s") * NUM_CORES + lax.axis_index("c")
    base_b = wid * ROWS_W
    gsem = (g0, g1)
    osem = (o0, o1)

    @pl.when(lax.axis_index("s") == 0)
    def _():
        pltpu.sync_copy(table_hbm, tab_s)

    pltpu.sync_copy(idx_hbm.at[pl.ds(wid * 2 * ROWS_W, 2 * ROWS_W)], idx_v)
    plsc.subcore_barrier()

    def fire_g(c, b):
        for j in range(K):
            r, h = j // 2, j % 2
            pltpu.async_copy(
                tab_s.at[idx_v.at[c * K + j]],
                rows_v.at[b].at[r].at[pl.ds(h * HALF, HALF)],
                gsem[b],
            )

    def wait_g(b):
        for _ in range(K):
            pltpu.make_async_copy(
                out_hbm.at[0].at[pl.ds(0, HALF)],
                rows_v.at[b].at[0].at[pl.ds(0, HALF)],
                gsem[b],
            ).wait()

    def fire_o(c, b):
        pltpu.async_copy(
            rows_v.at[b], out_hbm.at[pl.ds(base_b + c * CB, CB)], osem[b]
        )

    def drain_o(b):
        pltpu.make_async_copy(
            out_hbm.at[pl.ds(0, CB)], rows_v.at[b], osem[b]
        ).wait()

    def step(c, b):
        wait_g(1 - b)
        fire_o(c - 1, 1 - b)
        drain_o(b)
        fire_g(c, b)

    fire_g(0, 0)
    wait_g(0)
    fire_o(0, 0)
    fire_g(1, 1)

    def body(g, carry):
        step(2 * g, 0)
        step(2 * g + 1, 1)
        return carry

    lax.fori_loop(1, NCHUNK // 2, body, 0)

    wait_g(1)
    fire_o(NCHUNK - 1, 1)
    drain_o(0)
    drain_o(1)


def kernel(vectors, table):
    idx = vectors.reshape(2 * BATCH, HALF)
    return _embed(table, idx)


# speedup vs baseline: 2.8603x; 2.0484x over previous
import functools

import jax
import jax.numpy as jnp
from jax import lax
from jax.experimental import pallas as pl
from jax.experimental.pallas import tpu as pltpu
from jax.experimental.pallas import tpu_sc as plsc

BATCH = 4096
HIST = 200
EMBED_DIM = 64
VOCAB = 1001
NUM_CORES = 2
NUM_SUBCORES = 16
NW = NUM_CORES * NUM_SUBCORES
ROWS_W = BATCH // NW
HALF = HIST // 2
CB = 2
K = 2 * CB
NCHUNK = ROWS_W // CB

_mesh = plsc.VectorSubcoreMesh(core_axis_name="c", subcore_axis_name="s")


@functools.partial(
    pl.kernel,
    mesh=_mesh,
    out_type=jax.ShapeDtypeStruct((BATCH, HIST, 2 * EMBED_DIM), jnp.float32),
    scratch_types=[
        pltpu.VMEM((2 * ROWS_W, HALF), jnp.int32),
        pltpu.VMEM((2, CB, HIST, EMBED_DIM), jnp.float32),
        pltpu.VMEM_SHARED((VOCAB, EMBED_DIM), jnp.float32),
        pltpu.SemaphoreType.DMA,
        pltpu.SemaphoreType.DMA,
        pltpu.SemaphoreType.DMA,
        pltpu.SemaphoreType.DMA,
    ],
    compiler_params=pltpu.CompilerParams(use_tc_tiling_on_sc=False),
)
def _embed(table_hbm, idx_hbm, out_hbm, idx_v, rows_v, tab_s, g0, g1, o0, o1):
    wid = lax.axis_index("s") * NUM_CORES + lax.axis_index("c")
    base_b = wid * ROWS_W
    gsem = (g0, g1)
    osem = (o0, o1)

    @pl.when(lax.axis_index("s") == 0)
    def _():
        pltpu.sync_copy(table_hbm, tab_s)

    pltpu.sync_copy(idx_hbm.at[pl.ds(wid * 2 * ROWS_W, 2 * ROWS_W)], idx_v)
    plsc.subcore_barrier()

    def fire_g(c, b):
        for j in range(K):
            r, h = j // 2, j % 2
            pltpu.async_copy(
                tab_s.at[idx_v.at[c * K + j]],
                rows_v.at[b].at[r].at[pl.ds(h * HALF, HALF)],
                gsem[b],
            )

    def wait_g(b):
        for _ in range(K):
            pltpu.make_async_copy(
                out_hbm.at[0].at[pl.ds(0, HALF), pl.ds(0, EMBED_DIM)],
                rows_v.at[b].at[0].at[pl.ds(0, HALF)],
                gsem[b],
            ).wait()

    def fire_o(c, b):
        pltpu.async_copy(
            rows_v.at[b],
            out_hbm.at[pl.ds(base_b + c * CB, CB), :, pl.ds(0, EMBED_DIM)],
            osem[b],
        )

    def drain_o(b):
        pltpu.make_async_copy(
            out_hbm.at[pl.ds(0, CB), :, pl.ds(0, EMBED_DIM)],
            rows_v.at[b],
            osem[b],
        ).wait()

    def step(c, b):
        wait_g(1 - b)
        fire_o(c - 1, 1 - b)
        drain_o(b)
        fire_g(c, b)

    fire_g(0, 0)
    wait_g(0)
    fire_o(0, 0)
    fire_g(1, 1)

    def body(g, carry):
        step(2 * g, 0)
        step(2 * g + 1, 1)
        return carry

    lax.fori_loop(1, NCHUNK // 2, body, 0)

    wait_g(1)
    fire_o(NCHUNK - 1, 1)
    drain_o(0)
    drain_o(1)


def kernel(vectors, table):
    idx = vectors.reshape(2 * BATCH, HALF)
    out = _embed(table, idx)
    return out[:, :, :EMBED_DIM]


# speedup vs baseline: 2.8614x; 1.0004x over previous
import functools

import jax
import jax.numpy as jnp
from jax import lax
from jax.experimental import pallas as pl
from jax.experimental.pallas import tpu as pltpu
from jax.experimental.pallas import tpu_sc as plsc

BATCH = 4096
HIST = 200
EMBED_DIM = 64
VOCAB = 1001
NUM_CORES = 2
NUM_SUBCORES = 16
NW = NUM_CORES * NUM_SUBCORES
ROWS_W = BATCH // NW
HALF = HIST // 2
CB = 4
K = 2 * CB
NCHUNK = ROWS_W // CB

_mesh = plsc.VectorSubcoreMesh(core_axis_name="c", subcore_axis_name="s")


@functools.partial(
    pl.kernel,
    mesh=_mesh,
    out_type=jax.ShapeDtypeStruct((BATCH, HIST, 2 * EMBED_DIM), jnp.float32),
    scratch_types=[
        pltpu.VMEM((2, K, HALF), jnp.int32),
        pltpu.VMEM((2, CB, HIST, EMBED_DIM), jnp.float32),
        pltpu.VMEM_SHARED((VOCAB, EMBED_DIM), jnp.float32),
        pltpu.SemaphoreType.DMA,
        pltpu.SemaphoreType.DMA,
        pltpu.SemaphoreType.DMA,
        pltpu.SemaphoreType.DMA,
        pltpu.SemaphoreType.DMA,
        pltpu.SemaphoreType.DMA,
    ],
    compiler_params=pltpu.CompilerParams(use_tc_tiling_on_sc=False),
)
def _embed(table_hbm, idx_hbm, out_hbm, idx_v, rows_v, tab_s,
           g0, g1, o0, o1, i0, i1):
    wid = lax.axis_index("s") * NUM_CORES + lax.axis_index("c")
    base_b = wid * ROWS_W
    base_i = wid * 2 * ROWS_W
    gsem = (g0, g1)
    osem = (o0, o1)
    isem = (i0, i1)

    @pl.when(lax.axis_index("s") == 0)
    def _():
        pltpu.sync_copy(table_hbm, tab_s)

    def fire_i(c, b):
        cc = jnp.minimum(c, NCHUNK - 1)
        pltpu.async_copy(
            idx_hbm.at[pl.ds(base_i + cc * K, K)], idx_v.at[b], isem[b]
        )

    def wait_i(b):
        pltpu.make_async_copy(
            idx_hbm.at[pl.ds(0, K)], idx_v.at[b], isem[b]
        ).wait()

    def fire_g(c, b):
        for j in range(K):
            r, h = j // 2, j % 2
            pltpu.async_copy(
                tab_s.at[idx_v.at[b].at[j]],
                rows_v.at[b].at[r].at[pl.ds(h * HALF, HALF)],
                gsem[b],
            )

    def wait_g(b):
        for _ in range(K):
            pltpu.make_async_copy(
                out_hbm.at[0].at[pl.ds(0, HALF), pl.ds(0, EMBED_DIM)],
                rows_v.at[b].at[0].at[pl.ds(0, HALF)],
                gsem[b],
            ).wait()

    def fire_o(c, b):
        pltpu.async_copy(
            rows_v.at[b],
            out_hbm.at[pl.ds(base_b + c * CB, CB), :, pl.ds(0, EMBED_DIM)],
            osem[b],
        )

    def drain_o(b):
        pltpu.make_async_copy(
            out_hbm.at[pl.ds(0, CB), :, pl.ds(0, EMBED_DIM)],
            rows_v.at[b],
            osem[b],
        ).wait()

    plsc.subcore_barrier()

    def step(c, b):
        wait_g(1 - b)
        fire_o(c - 1, 1 - b)
        fire_i(c + 1, 1 - b)
        drain_o(b)
        wait_i(b)
        fire_g(c, b)

    fire_i(0, 0)
    wait_i(0)
    fire_g(0, 0)
    fire_i(1, 1)
    wait_g(0)
    fire_o(0, 0)
    fire_i(2, 0)
    wait_i(1)
    fire_g(1, 1)

    def body(g, carry):
        step(2 * g, 0)
        step(2 * g + 1, 1)
        return carry

    lax.fori_loop(1, NCHUNK // 2, body, 0)

    wait_g(1)
    fire_o(NCHUNK - 1, 1)
    wait_i(0)
    drain_o(0)
    drain_o(1)


def kernel(vectors, table):
    idx = vectors.reshape(2 * BATCH, HALF)
    out = _embed(table, idx)
    return out[:, :, :EMBED_DIM]
